# baseline (speedup 1.0000x reference)
import functools

import jax
import jax.numpy as jnp
from jax import lax
from jax.experimental import pallas as pl
from jax.experimental.pallas import tpu as pltpu
from jax.experimental.pallas import tpu_sc as plsc

D = 64
EPS = 1e-5
CHUNK = 128
LANES = 16
NVREG = D // LANES

_info = plsc.get_sparse_core_info()
_NC, _NS = _info.num_cores, _info.num_subcores
_NW = _NC * _NS


_GATHER_DNUMS = lax.GatherDimensionNumbers(
    offset_dims=(), collapsed_slice_dims=(0,), start_index_map=(0,))


def _lane_perm(v, idx):
    return lax.gather(v, idx[:, None], _GATHER_DNUMS, (1,),
                      mode=lax.GatherScatterMode.PROMISE_IN_BOUNDS)


def _allreduce_sum(v):
    for sh in (8, 4, 2, 1):
        idx = lax.iota(jnp.int32, LANES) ^ sh
        v = v + _lane_perm(v, idx)
    return v


def _rsqrt_vec(v):
    iv = lax.bitcast_convert_type(v, jnp.int32)
    seed = jnp.full((LANES,), 0x5F3759DF, jnp.int32) - lax.shift_right_logical(iv, 1)
    y = lax.bitcast_convert_type(seed, jnp.float32)
    half = v * 0.5
    for _ in range(3):
        y = y * (1.5 - half * y * y)
    return y


@functools.lru_cache(maxsize=None)
def _make_sc_kernel(BL):
    per_w = BL // _NW
    n_chunks = per_w // CHUNK
    mesh = plsc.VectorSubcoreMesh(core_axis_name="c", subcore_axis_name="s")

    @functools.partial(
        pl.kernel,
        out_type=jax.ShapeDtypeStruct((BL, D), jnp.float32),
        mesh=mesh,
        compiler_params=pltpu.CompilerParams(use_tc_tiling_on_sc=False),
        scratch_types=[
            pltpu.VMEM((n_chunks, CHUNK), jnp.int32),
            pltpu.VMEM((CHUNK, D), jnp.float32),
            pltpu.VMEM((D,), jnp.float32),
            pltpu.VMEM((D,), jnp.float32),
            pltpu.SemaphoreType.DMA,
        ],
    )
    def k(ids_hbm, table_hbm, gamma_hbm, beta_hbm, out_hbm,
          idx_v, rows_v, g_v, b_v, sem):
        wid = lax.axis_index("s") * _NC + lax.axis_index("c")
        base = wid * per_w
        pltpu.sync_copy(ids_hbm.at[wid], idx_v)
        pltpu.sync_copy(gamma_hbm, g_v)
        pltpu.sync_copy(beta_hbm, b_v)
        g = [g_v[pl.ds(LANES * t, LANES)] for t in range(NVREG)]
        b = [b_v[pl.ds(LANES * t, LANES)] for t in range(NVREG)]

        def chunk_body(j, carry):
            pltpu.async_copy(table_hbm.at[idx_v.at[j]], rows_v, sem).wait()

            def row_body(i, c2):
                x = [rows_v[i, pl.ds(LANES * t, LANES)] for t in range(NVREG)]
                s = (x[0] + x[1]) + (x[2] + x[3])
                q = (x[0] * x[0] + x[1] * x[1]) + (x[2] * x[2] + x[3] * x[3])
                mv = _allreduce_sum(s) * (1.0 / D)
                var = _allreduce_sum(q) * (1.0 / D) - mv * mv
                rv = _rsqrt_vec(var + EPS)
                for t in range(NVREG):
                    a = rv * g[t]
                    rows_v[i, pl.ds(LANES * t, LANES)] = (x[t] - mv) * a + b[t]
                return c2

            lax.fori_loop(0, CHUNK, row_body, 0)
            pltpu.sync_copy(rows_v, out_hbm.at[pl.ds(base + j * CHUNK, CHUNK)])
            return carry

        lax.fori_loop(0, n_chunks, chunk_body, 0)

    return k


def kernel(input_ids, table, gamma, beta):
    B, L = input_ids.shape
    BL = B * L
    per_w = BL // _NW
    ids = input_ids.reshape(-1).astype(jnp.int32)
    ids = ids.reshape(_NW, per_w // CHUNK, CHUNK)
    out = _make_sc_kernel(BL)(ids, table, gamma, beta)
    return out.reshape(B, L, D)


# speedup vs baseline: 1.1773x; 1.1773x over previous
import functools

import jax
import jax.numpy as jnp
from jax import lax
from jax.experimental import pallas as pl
from jax.experimental.pallas import tpu as pltpu
from jax.experimental.pallas import tpu_sc as plsc

D = 64
EPS = 1e-5
CHUNK = 128
NBUF = 4
LANES = 16
NVREG = D // LANES
UNROLL = 4

_info = plsc.get_sparse_core_info()
_NC, _NS = _info.num_cores, _info.num_subcores
_NW = _NC * _NS

_GATHER_DNUMS = lax.GatherDimensionNumbers(
    offset_dims=(), collapsed_slice_dims=(0,), start_index_map=(0,))


def _lane_perm(v, idx):
    return lax.gather(v, idx[:, None], _GATHER_DNUMS, (1,),
                      mode=lax.GatherScatterMode.PROMISE_IN_BOUNDS)


def _allreduce_sum(v):
    for sh in (8, 4, 2, 1):
        idx = lax.iota(jnp.int32, LANES) ^ sh
        v = v + _lane_perm(v, idx)
    return v


def _rsqrt_vec(v):
    iv = lax.bitcast_convert_type(v, jnp.int32)
    seed = jnp.full((LANES,), 0x5F3759DF, jnp.int32) - lax.shift_right_logical(iv, 1)
    y = lax.bitcast_convert_type(seed, jnp.float32)
    half = v * 0.5
    for _ in range(2):
        y = y * (1.5 - half * y * y)
    return y


@functools.lru_cache(maxsize=None)
def _make_sc_kernel(BL):
    per_w = BL // _NW
    n_chunks = per_w // CHUNK
    mesh = plsc.VectorSubcoreMesh(core_axis_name="c", subcore_axis_name="s")

    @functools.partial(
        pl.kernel,
        out_type=jax.ShapeDtypeStruct((BL, D), jnp.float32),
        mesh=mesh,
        compiler_params=pltpu.CompilerParams(use_tc_tiling_on_sc=False),
        scratch_types=[
            pltpu.VMEM((n_chunks, CHUNK), jnp.int32),
            pltpu.VMEM((NBUF, CHUNK, D), jnp.float32),
            pltpu.VMEM((D,), jnp.float32),
            pltpu.VMEM((D,), jnp.float32),
            pltpu.SemaphoreType.DMA((NBUF,)),
            pltpu.SemaphoreType.DMA((NBUF,)),
        ],
    )
    def k(ids_hbm, table_hbm, gamma_hbm, beta_hbm, out_hbm,
          idx_v, rows_v, g_v, b_v, gsem, osem):
        wid = lax.axis_index("s") * _NC + lax.axis_index("c")
        base = wid * per_w
        pltpu.sync_copy(ids_hbm.at[wid], idx_v)
        pltpu.sync_copy(gamma_hbm, g_v)
        pltpu.sync_copy(beta_hbm, b_v)
        g = [g_v[pl.ds(LANES * t, LANES)] for t in range(NVREG)]
        b = [b_v[pl.ds(LANES * t, LANES)] for t in range(NVREG)]

        def start_gather(j, buf):
            pltpu.async_copy(table_hbm.at[idx_v.at[j]], rows_v.at[buf],
                             gsem.at[buf])

        start_gather(0, 0)
        start_gather(1, 1)

        def ln_row(buf, i):
            x = [rows_v[buf, i, pl.ds(LANES * t, LANES)] for t in range(NVREG)]
            s = (x[0] + x[1]) + (x[2] + x[3])
            q = (x[0] * x[0] + x[1] * x[1]) + (x[2] * x[2] + x[3] * x[3])
            mv = _allreduce_sum(s) * (1.0 / D)
            var = _allreduce_sum(q) * (1.0 / D) - mv * mv
            rv = _rsqrt_vec(var + EPS)
            for t in range(NVREG):
                a = rv * g[t]
                rows_v[buf, i, pl.ds(LANES * t, LANES)] = (x[t] - mv) * a + b[t]

        def chunk_body(j, carry):
            buf = lax.rem(j, NBUF)
            buf2 = lax.rem(j + 2, NBUF)

            @pl.when(j + 2 < n_chunks)
            def _():
                @pl.when(j >= 2)
                def _():
                    pltpu.make_async_copy(
                        rows_v.at[buf2],
                        out_hbm.at[pl.ds(base + (j - 2) * CHUNK, CHUNK)],
                        osem.at[buf2]).wait()
                start_gather(j + 2, buf2)

            pltpu.make_async_copy(
                table_hbm.at[idx_v.at[j]], rows_v.at[buf], gsem.at[buf]).wait()

            def row_body(i, c2):
                for u in range(UNROLL):
                    ln_row(buf, i * UNROLL + u)
                return c2

            lax.fori_loop(0, CHUNK // UNROLL, row_body, 0, unroll=1)
            pltpu.async_copy(rows_v.at[buf],
                             out_hbm.at[pl.ds(base + j * CHUNK, CHUNK)],
                             osem.at[buf])
            return carry

        lax.fori_loop(0, n_chunks, chunk_body, 0)

        for j in (n_chunks - 2, n_chunks - 1):
            buf = j % NBUF
            pltpu.make_async_copy(
                rows_v.at[buf],
                out_hbm.at[pl.ds(base + j * CHUNK, CHUNK)],
                osem.at[buf]).wait()

    return k


def kernel(input_ids, table, gamma, beta):
    B, L = input_ids.shape
    BL = B * L
    per_w = BL // _NW
    ids = input_ids.reshape(-1).astype(jnp.int32)
    ids = ids.reshape(_NW, per_w // CHUNK, CHUNK)
    out = _make_sc_kernel(BL)(ids, table, gamma, beta)
    return out.reshape(B, L, D)
